# baseline (speedup 1.0000x reference)
import jax
import jax.numpy as jnp
from jax.experimental import pallas as pl
from jax.experimental.pallas import tpu as pltpu

D_MODEL = 2048
N_EXPERTS = 64
N_TOKENS = 8192
CAPACITY = 160
BLK = 1024
GRID = N_TOKENS // BLK
TRI = 256


def _router_body(x_ref, wt_ref, probs_ref, idx_ref, mask_ref, ovf_ref,
                 tri_ref, cnt_ref, acc_ref):
    i = pl.program_id(0)

    @pl.when(i == 0)
    def _init():
        r = jax.lax.broadcasted_iota(jnp.int32, (TRI, TRI), 0)
        c = jax.lax.broadcasted_iota(jnp.int32, (TRI, TRI), 1)
        tri_ref[...] = (r <= c).astype(jnp.bfloat16)
        cnt_ref[...] = jnp.zeros_like(cnt_ref)
        acc_ref[...] = jnp.zeros_like(acc_ref)

    x = x_ref[...]
    wt = wt_ref[...]
    logits = jnp.dot(x, wt, preferred_element_type=jnp.float32)
    m = jnp.max(logits, axis=-1, keepdims=True)
    ex = jnp.exp(logits - m)
    s = jnp.sum(ex, axis=-1, keepdims=True)
    probs = ex / s
    probs_ref[...] = probs

    probs_t = probs.T
    idx = jnp.argmax(probs_t, axis=0).astype(jnp.int32)
    idx_ref[...] = idx

    eq = (jax.lax.broadcasted_iota(jnp.int32, (N_EXPERTS, BLK), 0)
          == idx[None, :])
    one_hot_t = eq.astype(jnp.bfloat16)
    base = cnt_ref[...][:, 0:1]
    rows = TRI // 128
    for c in range(BLK // TRI):
        ohc = one_hot_t[:, c * TRI:(c + 1) * TRI]
        eqc = eq[:, c * TRI:(c + 1) * TRI]
        incl = jax.lax.dot_general(ohc, tri_ref[...],
                                   (((1,), (0,)), ((), ())),
                                   preferred_element_type=jnp.float32)
        posc = jnp.sum(jnp.where(eqc, incl + base, 0.0), axis=0) - 1.0
        keepc = posc < CAPACITY
        mask_ref[pl.ds(c * TRI, TRI)] = keepc.astype(jnp.int32)
        acc_ref[pl.ds(c * rows, rows), :] += (
            1.0 - keepc.astype(jnp.float32)).reshape(rows, 128)
        base = base + incl[:, TRI - 1:TRI]
    cnt_ref[...] = base + jnp.zeros_like(cnt_ref)

    @pl.when(i == GRID - 1)
    def _fin():
        ovf_ref[0, 0] = jnp.sum(acc_ref[...]).astype(jnp.int32)


def kernel(hidden, W):
    x = hidden.reshape(N_TOKENS, D_MODEL)
    wt = W.T
    probs, idx, mask_i32, ovf = pl.pallas_call(
        _router_body,
        grid=(GRID,),
        in_specs=[
            pl.BlockSpec((BLK, D_MODEL), lambda i: (i, 0)),
            pl.BlockSpec((D_MODEL, N_EXPERTS), lambda i: (0, 0)),
        ],
        out_specs=[
            pl.BlockSpec((BLK, N_EXPERTS), lambda i: (i, 0)),
            pl.BlockSpec((BLK,), lambda i: (i,)),
            pl.BlockSpec((BLK,), lambda i: (i,)),
            pl.BlockSpec(block_shape=(1, 1), index_map=lambda i: (0, 0),
                         memory_space=pltpu.SMEM),
        ],
        out_shape=[
            jax.ShapeDtypeStruct((N_TOKENS, N_EXPERTS), jnp.float32),
            jax.ShapeDtypeStruct((N_TOKENS,), jnp.int32),
            jax.ShapeDtypeStruct((N_TOKENS,), jnp.int32),
            jax.ShapeDtypeStruct((1, 1), jnp.int32),
        ],
        scratch_shapes=[
            pltpu.VMEM((TRI, TRI), jnp.bfloat16),
            pltpu.VMEM((N_EXPERTS, 128), jnp.float32),
            pltpu.VMEM((BLK // 128, 128), jnp.float32),
        ],
    )(x, wt)
    return probs, idx, mask_i32.astype(jnp.bool_), ovf[0, 0]


# speedup vs baseline: 1.0006x; 1.0006x over previous
import jax
import jax.numpy as jnp
from jax.experimental import pallas as pl
from jax.experimental.pallas import tpu as pltpu

D_MODEL = 2048
N_EXPERTS = 64
N_TOKENS = 8192
CAPACITY = 160
BLK = 1024
GRID = N_TOKENS // BLK
TRI = 256


def _router_body(xa_ref, xb_ref, wt_ref, probs_ref, idx_ref, mask_ref, ovf_ref,
                 tri_ref, cnt_ref, acc_ref):
    i = pl.program_id(0)

    @pl.when(i == 0)
    def _init():
        r = jax.lax.broadcasted_iota(jnp.int32, (TRI, TRI), 0)
        c = jax.lax.broadcasted_iota(jnp.int32, (TRI, TRI), 1)
        tri_ref[...] = (r <= c).astype(jnp.bfloat16)
        cnt_ref[...] = jnp.zeros_like(cnt_ref)
        acc_ref[...] = jnp.zeros_like(acc_ref)

    x = jnp.concatenate([xa_ref[...], xb_ref[...]], axis=0)
    wt = wt_ref[...]
    logits = jnp.dot(x, wt, preferred_element_type=jnp.float32)
    m = jnp.max(logits, axis=-1, keepdims=True)
    ex = jnp.exp(logits - m)
    s = jnp.sum(ex, axis=-1, keepdims=True)
    probs = ex / s
    probs_ref[...] = probs

    probs_t = probs.T
    idx = jnp.argmax(probs_t, axis=0).astype(jnp.int32)
    idx_ref[...] = idx

    eq = (jax.lax.broadcasted_iota(jnp.int32, (N_EXPERTS, BLK), 0)
          == idx[None, :])
    one_hot_t = eq.astype(jnp.bfloat16)
    base = cnt_ref[...][:, 0:1]
    rows = TRI // 128
    for c in range(BLK // TRI):
        ohc = one_hot_t[:, c * TRI:(c + 1) * TRI]
        eqc = eq[:, c * TRI:(c + 1) * TRI]
        incl = jax.lax.dot_general(ohc, tri_ref[...],
                                   (((1,), (0,)), ((), ())),
                                   preferred_element_type=jnp.float32)
        posc = jnp.sum(jnp.where(eqc, incl + base, 0.0), axis=0) - 1.0
        keepc = posc < CAPACITY
        mask_ref[pl.ds(c * TRI, TRI)] = keepc.astype(jnp.int32)
        acc_ref[pl.ds(c * rows, rows), :] += (
            1.0 - keepc.astype(jnp.float32)).reshape(rows, 128)
        base = base + incl[:, TRI - 1:TRI]
    cnt_ref[...] = base + jnp.zeros_like(cnt_ref)

    @pl.when(i == GRID - 1)
    def _fin():
        ovf_ref[0, 0] = jnp.sum(acc_ref[...]).astype(jnp.int32)


def kernel(hidden, W):
    x = hidden.reshape(N_TOKENS, D_MODEL)
    wt = W.T
    probs, idx, mask_i32, ovf = pl.pallas_call(
        _router_body,
        grid=(GRID,),
        in_specs=[
            pl.BlockSpec((BLK // 2, D_MODEL), lambda i: (2 * i, 0)),
            pl.BlockSpec((BLK // 2, D_MODEL), lambda i: (2 * i + 1, 0)),
            pl.BlockSpec((D_MODEL, N_EXPERTS), lambda i: (0, 0)),
        ],
        out_specs=[
            pl.BlockSpec((BLK, N_EXPERTS), lambda i: (i, 0)),
            pl.BlockSpec((BLK,), lambda i: (i,)),
            pl.BlockSpec((BLK,), lambda i: (i,)),
            pl.BlockSpec(block_shape=(1, 1), index_map=lambda i: (0, 0),
                         memory_space=pltpu.SMEM),
        ],
        out_shape=[
            jax.ShapeDtypeStruct((N_TOKENS, N_EXPERTS), jnp.float32),
            jax.ShapeDtypeStruct((N_TOKENS,), jnp.int32),
            jax.ShapeDtypeStruct((N_TOKENS,), jnp.int32),
            jax.ShapeDtypeStruct((1, 1), jnp.int32),
        ],
        scratch_shapes=[
            pltpu.VMEM((TRI, TRI), jnp.bfloat16),
            pltpu.VMEM((N_EXPERTS, 128), jnp.float32),
            pltpu.VMEM((BLK // 128, 128), jnp.float32),
        ],
    )(x, x, wt)
    return probs, idx, mask_i32.astype(jnp.bool_), ovf[0, 0]
